# baseline (speedup 1.0000x reference)
import functools
import jax
import jax.numpy as jnp
from jax import lax
from jax.experimental import pallas as pl
from jax.experimental.pallas import tpu as pltpu
from jax.experimental.pallas import tpu_sc as plsc

H = 1024
E = 8
T = 8192
BT = 512
BS = 512
PE = 2 * T + E * BS
P = T + PE
NT_SH = T // BS
NT = NT_SH + PE // BS
NC = 2
NS = 16
NW = NC * NS
LANES = 16

def _sc_mesh():
    return plsc.VectorSubcoreMesh(core_axis_name="c", subcore_axis_name="s")


def _router_tile(x_ref, gw_ref, gb_ref, tri_ref,
                 logits_ref, i1_ref, i2_ref, w1_ref, w2_ref,
                 r1_ref, r2_ref, offy_ref, te_ref, base_ref):
    i = pl.program_id(0)

    @pl.when(i == 0)
    def _():
        base_ref[...] = jnp.zeros((1, 16), jnp.float32)

    x = x_ref[...]
    logits = jnp.dot(x, gw_ref[...]) + gb_ref[...]
    logits_ref[...] = logits
    probs = jax.nn.softmax(logits, axis=-1)

    iota = lax.broadcasted_iota(jnp.int32, probs.shape, 1)
    v1 = jnp.max(probs, axis=-1, keepdims=True)
    i1 = jnp.min(jnp.where(probs == v1, iota, E), axis=-1, keepdims=True)
    one1 = iota == i1
    probs2 = jnp.where(one1, -jnp.inf, probs)
    v2 = jnp.max(probs2, axis=-1, keepdims=True)
    i2 = jnp.min(jnp.where(probs2 == v2, iota, E), axis=-1, keepdims=True)
    one2 = iota == i2
    denom = v1 + v2
    i1_ref[...] = i1
    i2_ref[...] = i2
    w1_ref[...] = v1 / denom
    w2_ref[...] = v2 / denom

    oh = jnp.concatenate([one1, one2], axis=0).astype(jnp.bfloat16)
    cum = jnp.dot(tri_ref[...], oh, preferred_element_type=jnp.float32)
    base = base_ref[0:1, 0:E]
    ohf = oh.astype(jnp.float32)
    rank = jnp.sum((cum + base) * ohf, axis=1, keepdims=True)
    r1_ref[...] = rank[:BT]
    r2_ref[...] = rank[BT:]
    new_base = base + jnp.sum(ohf, axis=0, keepdims=True)
    base_ref[0:1, 0:E] = new_base
    pc = jnp.ceil(new_base / BS) * BS
    inc = pc
    for sh in (1, 2, 4):
        inc = inc + jnp.concatenate(
            [jnp.zeros((1, sh), jnp.float32), inc[:, :-sh]], axis=1)
    offy = inc - pc + T
    zpad = jnp.zeros((1, 16 - E), jnp.float32)
    offy_ref[...] = jnp.concatenate([offy, zpad], axis=1).astype(jnp.int32)
    jt = lax.broadcasted_iota(jnp.int32, (1, 64), 1)
    ts = ((jt - NT_SH) * BS).astype(jnp.float32)
    acc = jnp.zeros((1, 64), jnp.int32)
    for e in range(E):
        acc = acc + jnp.where(ts >= inc[:, e:e + 1], 1, 0)
    te_ref[...] = jnp.where(jt < NT_SH, E, jnp.minimum(acc, E))


def _run_router(hs, gate_w, gate_b, tri):
    n = T // BT
    return pl.pallas_call(
        _router_tile,
        grid=(n,),
        in_specs=[
            pl.BlockSpec((BT, H), lambda i: (i, 0)),
            pl.BlockSpec((H, E), lambda i: (0, 0)),
            pl.BlockSpec((1, E), lambda i: (0, 0)),
            pl.BlockSpec((2 * BT, 2 * BT), lambda i: (0, 0)),
        ],
        out_specs=[
            pl.BlockSpec((BT, E), lambda i: (i, 0)),
            pl.BlockSpec((BT, 1), lambda i: (i, 0)),
            pl.BlockSpec((BT, 1), lambda i: (i, 0)),
            pl.BlockSpec((BT, 1), lambda i: (i, 0)),
            pl.BlockSpec((BT, 1), lambda i: (i, 0)),
            pl.BlockSpec((BT, 1), lambda i: (i, 0)),
            pl.BlockSpec((BT, 1), lambda i: (i, 0)),
            pl.BlockSpec((1, 16), lambda i: (0, 0)),
            pl.BlockSpec((1, 64), lambda i: (0, 0)),
        ],
        out_shape=[
            jax.ShapeDtypeStruct((T, E), jnp.float32),
            jax.ShapeDtypeStruct((T, 1), jnp.int32),
            jax.ShapeDtypeStruct((T, 1), jnp.int32),
            jax.ShapeDtypeStruct((T, 1), jnp.float32),
            jax.ShapeDtypeStruct((T, 1), jnp.float32),
            jax.ShapeDtypeStruct((T, 1), jnp.float32),
            jax.ShapeDtypeStruct((T, 1), jnp.float32),
            jax.ShapeDtypeStruct((1, 16), jnp.int32),
            jax.ShapeDtypeStruct((1, 64), jnp.int32),
        ],
        scratch_shapes=[pltpu.VMEM((1, 16), jnp.float32)],
        compiler_params=pltpu.CompilerParams(
            dimension_semantics=("arbitrary",)),
    )(hs, gate_w, gate_b.reshape(1, E), tri)


def _run_dispatch(i1, i2, r1, r2, w1, w2, offy16):
    k = functools.partial(
        pl.kernel,
        out_type=[
            jax.ShapeDtypeStruct((T,), jnp.int32),
            jax.ShapeDtypeStruct((T,), jnp.int32),
            jax.ShapeDtypeStruct((PE,), jnp.int32),
            jax.ShapeDtypeStruct((PE,), jnp.float32),
        ],
        mesh=_sc_mesh(),
        compiler_params=pltpu.CompilerParams(needs_layout_passes=False),
        scratch_types=[
            pltpu.VMEM((T,), jnp.int32), pltpu.VMEM((T,), jnp.int32),
            pltpu.VMEM((T,), jnp.float32), pltpu.VMEM((T,), jnp.float32),
            pltpu.VMEM((T,), jnp.float32), pltpu.VMEM((T,), jnp.float32),
            pltpu.VMEM((16,), jnp.int32),
            pltpu.VMEM((T,), jnp.int32), pltpu.VMEM((T,), jnp.int32),
            pltpu.VMEM((PE,), jnp.int32), pltpu.VMEM((PE,), jnp.float32),
        ],
    )(_dispatch_kernel)
    return k(i1, i2, r1, r2, w1, w2, offy16)


def _dispatch_kernel(i1_h, i2_h, r1_h, r2_h, w1_h, w2_h, offy_h,
                     pos1_h, pos2_h, stok_h, wslot_h,
                     i1v, i2v, r1v, r2v, w1v, w2v,
                     offy, p1v, p2v, stv, wsv):
    wid = lax.axis_index("s") * NC + lax.axis_index("c")

    @pl.when(wid == 0)
    def _():
        pltpu.sync_copy(i1_h, i1v)
        pltpu.sync_copy(i2_h, i2v)
        pltpu.sync_copy(r1_h, r1v)
        pltpu.sync_copy(r2_h, r2v)
        pltpu.sync_copy(w1_h, w1v)
        pltpu.sync_copy(w2_h, w2v)
        pltpu.sync_copy(offy_h, offy)

        def initb(k, c):
            stv[pl.ds(k * 16, 16)] = jnp.zeros((16,), jnp.int32)
            wsv[pl.ds(k * 16, 16)] = jnp.zeros((16,), jnp.float32)
            return c
        lax.fori_loop(0, PE // 16, initb, 0)

        def body(k, c):
            s = k * 16
            tok = lax.broadcasted_iota(jnp.int32, (16,), 0) + s
            for iv, rv, wv, posv in ((i1v, r1v, w1v, p1v),
                                     (i2v, r2v, w2v, p2v)):
                e = iv[pl.ds(s, 16)]
                off = plsc.load_gather(offy, [e])
                r = rv[pl.ds(s, 16)].astype(jnp.int32)
                p = off + r
                posv[pl.ds(s, 16)] = p
                l = p - T
                plsc.store_scatter(stv, [l], tok)
                plsc.store_scatter(wsv, [l], wv[pl.ds(s, 16)])
            return c
        lax.fori_loop(0, T // 16, body, 0)

        pltpu.sync_copy(p1v, pos1_h)
        pltpu.sync_copy(p2v, pos2_h)
        pltpu.sync_copy(stv, stok_h)
        pltpu.sync_copy(wsv, wslot_h)


_C_CH = 128
_C_PER_W = PE // NW


def _run_gather(xi, stok):
    k = functools.partial(
        pl.kernel,
        out_type=jax.ShapeDtypeStruct((PE, H // 2), jnp.int32),
        mesh=_sc_mesh(),
        compiler_params=pltpu.CompilerParams(needs_layout_passes=False),
        scratch_types=[
            pltpu.VMEM((_C_CH,), jnp.int32),
            pltpu.VMEM((_C_CH, H // 2), jnp.int32),
            pltpu.SemaphoreType.DMA,
        ],
    )(_gather_kernel)
    return k(xi, stok)


def _gather_kernel(xi_h, stok_h, xd_h, idxv, buf, sem):
    wid = lax.axis_index("s") * NC + lax.axis_index("c")
    base = wid * _C_PER_W
    for c in range(_C_PER_W // _C_CH):
        row0 = base + c * _C_CH
        pltpu.sync_copy(stok_h.at[pl.ds(row0, _C_CH)], idxv)
        pltpu.async_copy(xi_h.at[idxv], buf, sem).wait()
        pltpu.sync_copy(buf, xd_h.at[pl.ds(row0, _C_CH)])


def _group_mm_tile(te_ref, x_ref, xd_ref, w_ref, b_ref, ws_ref, y_ref):
    i = pl.program_id(0)

    @pl.when(i < NT_SH)
    def _():
        y = jnp.dot(x_ref[...], w_ref[0], preferred_element_type=jnp.float32)
        y_ref[...] = (y + b_ref[0]).astype(jnp.bfloat16)

    @pl.when(i >= NT_SH)
    def _():
        y = jnp.dot(xd_ref[...], w_ref[0], preferred_element_type=jnp.float32)
        y_ref[...] = (ws_ref[...] * (y + b_ref[0])).astype(jnp.bfloat16)


def _run_group_mm(te, hs_bf, xd, wcat, bcat, wslot):
    grid_spec = pltpu.PrefetchScalarGridSpec(
        num_scalar_prefetch=1,
        grid=(NT,),
        in_specs=[
            pl.BlockSpec((BS, H), lambda i, te: (jnp.minimum(i, NT_SH - 1), 0)),
            pl.BlockSpec((BS, H), lambda i, te: (jnp.maximum(i - NT_SH, 0), 0)),
            pl.BlockSpec((1, H, H), lambda i, te: (te[i], 0, 0)),
            pl.BlockSpec((1, 1, H), lambda i, te: (te[i], 0, 0)),
            pl.BlockSpec((BS, 1), lambda i, te: (jnp.maximum(i - NT_SH, 0), 0)),
        ],
        out_specs=pl.BlockSpec((BS, H), lambda i, te: (i, 0)),
    )
    return pl.pallas_call(
        _group_mm_tile,
        grid_spec=grid_spec,
        out_shape=jax.ShapeDtypeStruct((P, H), jnp.bfloat16),
        compiler_params=pltpu.CompilerParams(
            dimension_semantics=("arbitrary",)),
    )(te, hs_bf, xd, wcat, bcat, wslot)


_E_CH = 32
_E_PER_W = T // NW


def _run_combine(yi, pos1, pos2):
    k = functools.partial(
        pl.kernel,
        out_type=jax.ShapeDtypeStruct((T, H // 2), jnp.int32),
        mesh=_sc_mesh(),
        compiler_params=pltpu.CompilerParams(needs_layout_passes=False),
        scratch_types=[
            pltpu.VMEM((_E_CH,), jnp.int32),
            pltpu.VMEM((_E_CH,), jnp.int32),
            pltpu.VMEM((_E_CH, H // 2), jnp.int32),
            pltpu.VMEM((_E_CH, H // 2), jnp.int32),
            pltpu.VMEM((_E_CH, H // 2), jnp.int32),
            pltpu.SemaphoreType.DMA,
        ],
    )(_combine_kernel)
    return k(yi, pos1, pos2)


def _combine_kernel(yi_h, pos1_h, pos2_h, out_h, p1v, p2v, b0, b1, b2, sem):
    wid = lax.axis_index("s") * NC + lax.axis_index("c")
    tok0 = wid * _E_PER_W
    for c in range(_E_PER_W // _E_CH):
        t0 = tok0 + c * _E_CH
        pltpu.sync_copy(pos1_h.at[pl.ds(t0, _E_CH)], p1v)
        pltpu.sync_copy(pos2_h.at[pl.ds(t0, _E_CH)], p2v)
        pltpu.async_copy(yi_h.at[p1v], b1, sem).wait()
        pltpu.async_copy(yi_h.at[p2v], b2, sem).wait()
        pltpu.sync_copy(yi_h.at[pl.ds(t0, _E_CH)], b0)

        def row(r, carry):
            for g in range(H // 2 // 16):
                a = plsc.bitcast(b0[r, pl.ds(g * 16, 16)], jnp.bfloat16)
                u = plsc.bitcast(b1[r, pl.ds(g * 16, 16)], jnp.bfloat16)
                v = plsc.bitcast(b2[r, pl.ds(g * 16, 16)], jnp.bfloat16)
                b0[r, pl.ds(g * 16, 16)] = plsc.bitcast(a + u + v, jnp.int32)
            return carry
        lax.fori_loop(0, _E_CH, row, 0)
        pltpu.sync_copy(b0, out_h.at[pl.ds(t0, _E_CH)])


def kernel(x, gate_w, gate_b, expert_w, expert_b, shared_w, shared_b):
    b, s, h = x.shape
    hs = x.reshape(T, H)
    hs_bf = hs.astype(jnp.bfloat16)
    hsi = lax.bitcast_convert_type(
        hs_bf.reshape(T, H // 2, 2), jnp.int32)
    wcat = jnp.concatenate(
        [expert_w, (shared_w[0] + shared_w[1])[None]], axis=0
    ).astype(jnp.bfloat16)
    bcat = jnp.concatenate(
        [expert_b, (shared_b[0] + shared_b[1])[None]], axis=0)
    tri = jnp.tril(jnp.ones((2 * BT, 2 * BT), jnp.float32), -1).astype(
        jnp.bfloat16)

    (logits, i1, i2, w1, w2, r1, r2, offy16, te64) = _run_router(
        hs, gate_w, gate_b, tri)

    pos1, pos2, stok, wslot = _run_dispatch(
        i1.reshape(T), i2.reshape(T), r1.reshape(T), r2.reshape(T),
        w1.reshape(T), w2.reshape(T), offy16.reshape(16))
    te = te64.reshape(64)

    xdi = _run_gather(hsi, stok)
    xd = lax.bitcast_convert_type(
        xdi.reshape(PE, H // 2, 1), jnp.bfloat16).reshape(PE, H)

    y = _run_group_mm(te, hs_bf, xd, wcat,
                      bcat.reshape(E + 1, 1, H), wslot.reshape(PE, 1))

    yi = lax.bitcast_convert_type(y.reshape(P, H // 2, 2), jnp.int32)
    outi = _run_combine(yi, pos1, pos2)
    out_bf = lax.bitcast_convert_type(
        outi.reshape(T, H // 2, 1), jnp.bfloat16).reshape(T, H)

    out = out_bf.reshape(b, s, h).astype(jnp.float32)
    return out, logits


# speedup vs baseline: 2.9539x; 2.9539x over previous
import functools
import jax
import jax.numpy as jnp
from jax import lax
from jax.experimental import pallas as pl
from jax.experimental.pallas import tpu as pltpu
from jax.experimental.pallas import tpu_sc as plsc

H = 1024
E = 8
T = 8192
BT = 512
BS = 512
PE = 2 * T + E * BS
P = T + PE
NT_SH = T // BS
NT = NT_SH + PE // BS
NC = 2
NS = 16
NW = NC * NS
LANES = 16

def _sc_mesh():
    return plsc.VectorSubcoreMesh(core_axis_name="c", subcore_axis_name="s")


def _router_tile(x_ref, gw_ref, gb_ref, tri_ref,
                 logits_ref, i1_ref, i2_ref, w1_ref, w2_ref,
                 r1_ref, r2_ref, offy_ref, te_ref, base_ref):
    i = pl.program_id(0)

    @pl.when(i == 0)
    def _():
        base_ref[...] = jnp.zeros((1, 16), jnp.float32)

    x = x_ref[...]
    logits = jnp.dot(x, gw_ref[...]) + gb_ref[...]
    logits_ref[...] = logits
    probs = jax.nn.softmax(logits, axis=-1)

    iota = lax.broadcasted_iota(jnp.int32, probs.shape, 1)
    v1 = jnp.max(probs, axis=-1, keepdims=True)
    i1 = jnp.min(jnp.where(probs == v1, iota, E), axis=-1, keepdims=True)
    one1 = iota == i1
    probs2 = jnp.where(one1, -jnp.inf, probs)
    v2 = jnp.max(probs2, axis=-1, keepdims=True)
    i2 = jnp.min(jnp.where(probs2 == v2, iota, E), axis=-1, keepdims=True)
    one2 = iota == i2
    denom = v1 + v2
    i1_ref[...] = i1
    i2_ref[...] = i2
    w1_ref[...] = v1 / denom
    w2_ref[...] = v2 / denom

    oh = jnp.concatenate([one1, one2], axis=0).astype(jnp.bfloat16)
    cum = jnp.dot(tri_ref[...], oh, preferred_element_type=jnp.float32)
    base = base_ref[0:1, 0:E]
    ohf = oh.astype(jnp.float32)
    rank = jnp.sum((cum + base) * ohf, axis=1, keepdims=True)
    r1_ref[...] = rank[:BT]
    r2_ref[...] = rank[BT:]
    new_base = base + jnp.sum(ohf, axis=0, keepdims=True)
    base_ref[0:1, 0:E] = new_base
    pc = jnp.ceil(new_base / BS) * BS
    inc = pc
    for sh in (1, 2, 4):
        inc = inc + jnp.concatenate(
            [jnp.zeros((1, sh), jnp.float32), inc[:, :-sh]], axis=1)
    offy = inc - pc + T
    zpad = jnp.zeros((1, 16 - E), jnp.float32)
    offy_ref[...] = jnp.concatenate([offy, zpad], axis=1).astype(jnp.int32)
    jt = lax.broadcasted_iota(jnp.int32, (1, 64), 1)
    ts = ((jt - NT_SH) * BS).astype(jnp.float32)
    acc = jnp.zeros((1, 64), jnp.int32)
    for e in range(E):
        acc = acc + jnp.where(ts >= inc[:, e:e + 1], 1, 0)
    te_ref[...] = jnp.where(jt < NT_SH, E, jnp.minimum(acc, E))


def _run_router(hs, gate_w, gate_b, tri):
    n = T // BT
    return pl.pallas_call(
        _router_tile,
        grid=(n,),
        in_specs=[
            pl.BlockSpec((BT, H), lambda i: (i, 0)),
            pl.BlockSpec((H, E), lambda i: (0, 0)),
            pl.BlockSpec((1, E), lambda i: (0, 0)),
            pl.BlockSpec((2 * BT, 2 * BT), lambda i: (0, 0)),
        ],
        out_specs=[
            pl.BlockSpec((BT, E), lambda i: (i, 0)),
            pl.BlockSpec((BT, 1), lambda i: (i, 0)),
            pl.BlockSpec((BT, 1), lambda i: (i, 0)),
            pl.BlockSpec((BT, 1), lambda i: (i, 0)),
            pl.BlockSpec((BT, 1), lambda i: (i, 0)),
            pl.BlockSpec((BT, 1), lambda i: (i, 0)),
            pl.BlockSpec((BT, 1), lambda i: (i, 0)),
            pl.BlockSpec((1, 16), lambda i: (0, 0)),
            pl.BlockSpec((1, 64), lambda i: (0, 0)),
        ],
        out_shape=[
            jax.ShapeDtypeStruct((T, E), jnp.float32),
            jax.ShapeDtypeStruct((T, 1), jnp.int32),
            jax.ShapeDtypeStruct((T, 1), jnp.int32),
            jax.ShapeDtypeStruct((T, 1), jnp.float32),
            jax.ShapeDtypeStruct((T, 1), jnp.float32),
            jax.ShapeDtypeStruct((T, 1), jnp.float32),
            jax.ShapeDtypeStruct((T, 1), jnp.float32),
            jax.ShapeDtypeStruct((1, 16), jnp.int32),
            jax.ShapeDtypeStruct((1, 64), jnp.int32),
        ],
        scratch_shapes=[pltpu.VMEM((1, 16), jnp.float32)],
        compiler_params=pltpu.CompilerParams(
            dimension_semantics=("arbitrary",)),
    )(hs, gate_w, gate_b.reshape(1, E), tri)


def _run_dispatch(i1, i2, r1, r2, w1, w2, offy16):
    k = functools.partial(
        pl.kernel,
        out_type=[
            jax.ShapeDtypeStruct((T,), jnp.int32),
            jax.ShapeDtypeStruct((T,), jnp.int32),
            jax.ShapeDtypeStruct((PE,), jnp.int32),
            jax.ShapeDtypeStruct((PE,), jnp.float32),
        ],
        mesh=_sc_mesh(),
        compiler_params=pltpu.CompilerParams(needs_layout_passes=False),
        scratch_types=[
            pltpu.VMEM((T,), jnp.int32), pltpu.VMEM((T,), jnp.int32),
            pltpu.VMEM((T,), jnp.float32), pltpu.VMEM((T,), jnp.float32),
            pltpu.VMEM((T,), jnp.float32), pltpu.VMEM((T,), jnp.float32),
            pltpu.VMEM((16,), jnp.int32),
            pltpu.VMEM((T,), jnp.int32), pltpu.VMEM((T,), jnp.int32),
            pltpu.VMEM((PE,), jnp.int32), pltpu.VMEM((PE,), jnp.float32),
        ],
    )(_dispatch_kernel)
    return k(i1, i2, r1, r2, w1, w2, offy16)


def _dispatch_kernel(i1_h, i2_h, r1_h, r2_h, w1_h, w2_h, offy_h,
                     pos1_h, pos2_h, stok_h, wslot_h,
                     i1v, i2v, r1v, r2v, w1v, w2v,
                     offy, p1v, p2v, stv, wsv):
    wid = lax.axis_index("s") * NC + lax.axis_index("c")

    @pl.when(wid == 0)
    def _():
        pltpu.sync_copy(i1_h, i1v)
        pltpu.sync_copy(i2_h, i2v)
        pltpu.sync_copy(r1_h, r1v)
        pltpu.sync_copy(r2_h, r2v)
        pltpu.sync_copy(w1_h, w1v)
        pltpu.sync_copy(w2_h, w2v)
        pltpu.sync_copy(offy_h, offy)

        @plsc.parallel_loop(0, PE, step=16, unroll=8)
        def _init(s):
            stv[pl.ds(s, 16)] = jnp.zeros((16,), jnp.int32)
            wsv[pl.ds(s, 16)] = jnp.zeros((16,), jnp.float32)

        @plsc.parallel_loop(0, T, step=16, unroll=8)
        def _scatter(s):
            tok = lax.broadcasted_iota(jnp.int32, (16,), 0) + s
            for iv, rv, wv, posv in ((i1v, r1v, w1v, p1v),
                                     (i2v, r2v, w2v, p2v)):
                e = iv[pl.ds(s, 16)]
                off = plsc.load_gather(offy, [e])
                r = rv[pl.ds(s, 16)].astype(jnp.int32)
                p = off + r
                posv[pl.ds(s, 16)] = p
                l = p - T
                plsc.store_scatter(stv, [l], tok)
                plsc.store_scatter(wsv, [l], wv[pl.ds(s, 16)])

        pltpu.sync_copy(p1v, pos1_h)
        pltpu.sync_copy(p2v, pos2_h)
        pltpu.sync_copy(stv, stok_h)
        pltpu.sync_copy(wsv, wslot_h)


_C_CH = 64
_C_PER_W = PE // NW


def _run_gather(xf, stok):
    k = functools.partial(
        pl.kernel,
        out_type=jax.ShapeDtypeStruct((PE, H), jnp.float32),
        mesh=_sc_mesh(),
        compiler_params=pltpu.CompilerParams(needs_layout_passes=False),
        scratch_types=[
            pltpu.VMEM((_C_CH,), jnp.int32),
            pltpu.VMEM((_C_CH, H), jnp.float32),
            pltpu.SemaphoreType.DMA,
        ],
    )(_gather_kernel)
    return k(xf, stok)


def _gather_kernel(xf_h, stok_h, xd_h, idxv, buf, sem):
    wid = lax.axis_index("s") * NC + lax.axis_index("c")
    base = wid * _C_PER_W
    for c in range(_C_PER_W // _C_CH):
        row0 = base + c * _C_CH
        pltpu.sync_copy(stok_h.at[pl.ds(row0, _C_CH)], idxv)
        pltpu.async_copy(xf_h.at[idxv], buf, sem).wait()
        pltpu.sync_copy(buf, xd_h.at[pl.ds(row0, _C_CH)])


def _group_mm_tile(te_ref, x_ref, xd_ref, w_ref, b_ref, ws_ref, y_ref):
    i = pl.program_id(0)

    @pl.when(i < NT_SH)
    def _():
        xb = x_ref[...].astype(jnp.bfloat16)
        y = jnp.dot(xb, w_ref[0], preferred_element_type=jnp.float32)
        y_ref[...] = y + b_ref[0]

    @pl.when(i >= NT_SH)
    def _():
        xb = xd_ref[...].astype(jnp.bfloat16)
        y = jnp.dot(xb, w_ref[0], preferred_element_type=jnp.float32)
        y_ref[...] = ws_ref[...] * (y + b_ref[0])


def _run_group_mm(te, hs, xd, wcat, bcat, wslot):
    grid_spec = pltpu.PrefetchScalarGridSpec(
        num_scalar_prefetch=1,
        grid=(NT,),
        in_specs=[
            pl.BlockSpec((BS, H), lambda i, te: (jnp.minimum(i, NT_SH - 1), 0)),
            pl.BlockSpec((BS, H), lambda i, te: (jnp.maximum(i - NT_SH, 0), 0)),
            pl.BlockSpec((1, H, H), lambda i, te: (te[i], 0, 0)),
            pl.BlockSpec((1, 1, H), lambda i, te: (te[i], 0, 0)),
            pl.BlockSpec((BS, 1), lambda i, te: (jnp.maximum(i - NT_SH, 0), 0)),
        ],
        out_specs=pl.BlockSpec((BS, H), lambda i, te: (i, 0)),
    )
    return pl.pallas_call(
        _group_mm_tile,
        grid_spec=grid_spec,
        out_shape=jax.ShapeDtypeStruct((P, H), jnp.float32),
        compiler_params=pltpu.CompilerParams(
            dimension_semantics=("arbitrary",)),
    )(te, hs, xd, wcat, bcat, wslot)


_E_CH = 32
_E_PER_W = T // NW


def _run_combine(yf, pos1, pos2):
    k = functools.partial(
        pl.kernel,
        out_type=jax.ShapeDtypeStruct((T, H), jnp.float32),
        mesh=_sc_mesh(),
        compiler_params=pltpu.CompilerParams(needs_layout_passes=False),
        scratch_types=[
            pltpu.VMEM((_E_CH,), jnp.int32),
            pltpu.VMEM((_E_CH,), jnp.int32),
            pltpu.VMEM((_E_CH, H), jnp.float32),
            pltpu.VMEM((_E_CH, H), jnp.float32),
            pltpu.VMEM((_E_CH, H), jnp.float32),
            pltpu.SemaphoreType.DMA,
        ],
    )(_combine_kernel)
    return k(yf, pos1, pos2)


def _combine_kernel(yf_h, pos1_h, pos2_h, out_h, p1v, p2v, b0, b1, b2, sem):
    wid = lax.axis_index("s") * NC + lax.axis_index("c")
    tok0 = wid * _E_PER_W
    for c in range(_E_PER_W // _E_CH):
        t0 = tok0 + c * _E_CH
        pltpu.sync_copy(pos1_h.at[pl.ds(t0, _E_CH)], p1v)
        pltpu.sync_copy(pos2_h.at[pl.ds(t0, _E_CH)], p2v)
        c1 = pltpu.async_copy(yf_h.at[p1v], b1, sem)
        c2 = pltpu.async_copy(yf_h.at[p2v], b2, sem)
        c3 = pltpu.async_copy(yf_h.at[pl.ds(t0, _E_CH)], b0, sem)
        c1.wait()
        c2.wait()
        c3.wait()

        @plsc.parallel_loop(0, _E_CH, step=1, unroll=1)
        def _row(r):
            for g in range(H // 16):
                sl = pl.ds(g * 16, 16)
                b0[r, sl] = b0[r, sl] + b1[r, sl] + b2[r, sl]
        pltpu.sync_copy(b0, out_h.at[pl.ds(t0, _E_CH)])


def kernel(x, gate_w, gate_b, expert_w, expert_b, shared_w, shared_b):
    b, s, h = x.shape
    hs = x.reshape(T, H)
    wcat = jnp.concatenate(
        [expert_w, (shared_w[0] + shared_w[1])[None]], axis=0
    ).astype(jnp.bfloat16)
    bcat = jnp.concatenate(
        [expert_b, (shared_b[0] + shared_b[1])[None]], axis=0)
    tri = jnp.tril(jnp.ones((2 * BT, 2 * BT), jnp.float32), -1).astype(
        jnp.bfloat16)

    (logits, i1, i2, w1, w2, r1, r2, offy16, te64) = _run_router(
        hs, gate_w, gate_b, tri)

    pos1, pos2, stok, wslot = _run_dispatch(
        i1.reshape(T), i2.reshape(T), r1.reshape(T), r2.reshape(T),
        w1.reshape(T), w2.reshape(T), offy16.reshape(16))
    te = te64.reshape(64)

    xd = _run_gather(hs, stok)

    y = _run_group_mm(te, hs, xd, wcat,
                      bcat.reshape(E + 1, 1, H), wslot.reshape(PE, 1))

    out = _run_combine(y, pos1, pos2).reshape(b, s, h)
    return out, logits


# speedup vs baseline: 8.5060x; 2.8796x over previous
import jax
import jax.numpy as jnp
from jax.experimental import pallas as pl
from jax.experimental.pallas import tpu as pltpu

_HIDDEN = 1024
_E = 8
_BT = 1024


def _moe_tile(x_ref, gw_ref, gb_ref, wcat_ref, eb_ref, sb_ref,
              out_ref, logits_ref):
    x = x_ref[...]

    logits = jnp.dot(x, gw_ref[...]) + gb_ref[...]
    logits_ref[...] = logits
    probs = jax.nn.softmax(logits, axis=-1)

    iota = jax.lax.broadcasted_iota(jnp.int32, probs.shape, 1)
    v1 = jnp.max(probs, axis=-1, keepdims=True)
    i1 = jnp.min(jnp.where(probs == v1, iota, _E), axis=-1, keepdims=True)
    one1 = iota == i1
    probs2 = jnp.where(one1, -jnp.inf, probs)
    v2 = jnp.max(probs2, axis=-1, keepdims=True)
    i2 = jnp.min(jnp.where(probs2 == v2, iota, _E), axis=-1, keepdims=True)
    one2 = iota == i2
    denom = v1 + v2
    combine = jnp.where(one1, v1 / denom, 0.0) + jnp.where(one2, v2 / denom, 0.0)
    combine = combine.astype(jnp.float32)

    acc = jnp.dot(combine, eb_ref[...], preferred_element_type=jnp.float32)
    acc += sb_ref[0:1, :] + sb_ref[1:2, :]

    xb = x.astype(jnp.bfloat16)
    cb = combine.astype(jnp.bfloat16)
    acc += jnp.dot(xb, wcat_ref[_E], preferred_element_type=jnp.float32)
    for e in range(_E):
        xe = xb * cb[:, e:e + 1]
        acc += jnp.dot(xe, wcat_ref[e], preferred_element_type=jnp.float32)

    out_ref[...] = acc


def kernel(x, gate_w, gate_b, expert_w, expert_b, shared_w, shared_b):
    b, s, h = x.shape
    hs = x.reshape(-1, h)
    t = hs.shape[0]
    wcat = jnp.concatenate(
        [expert_w, (shared_w[0] + shared_w[1])[None]], axis=0
    ).astype(jnp.bfloat16)

    grid = (t // _BT,)
    out, logits = pl.pallas_call(
        _moe_tile,
        grid=grid,
        in_specs=[
            pl.BlockSpec((_BT, h), lambda i: (i, 0)),
            pl.BlockSpec((h, _E), lambda i: (0, 0)),
            pl.BlockSpec((1, _E), lambda i: (0, 0)),
            pl.BlockSpec((_E + 1, h, h), lambda i: (0, 0, 0)),
            pl.BlockSpec((_E, h), lambda i: (0, 0)),
            pl.BlockSpec((2, h), lambda i: (0, 0)),
        ],
        out_specs=[
            pl.BlockSpec((_BT, h), lambda i: (i, 0)),
            pl.BlockSpec((_BT, _E), lambda i: (i, 0)),
        ],
        out_shape=[
            jax.ShapeDtypeStruct((t, h), jnp.float32),
            jax.ShapeDtypeStruct((t, _E), jnp.float32),
        ],
        compiler_params=pltpu.CompilerParams(
            dimension_semantics=("arbitrary",),
        ),
    )(hs, gate_w, gate_b.reshape(1, _E), wcat, expert_b, shared_b)
    return out.reshape(b, s, h), logits


# speedup vs baseline: 8.7561x; 1.0294x over previous
import jax
import jax.numpy as jnp
from jax.experimental import pallas as pl
from jax.experimental.pallas import tpu as pltpu

_HIDDEN = 1024
_E = 8
_BT = 1024


def _moe_tile(x_ref, gw_ref, gb_ref, ew_hbm, sw_hbm, eb_ref, sb_ref,
              out_ref, logits_ref, wcat_ref, stage_ref, sem):
    i = pl.program_id(0)

    @pl.when(i == 0)
    def _():
        for e in range(_E):
            pltpu.make_async_copy(ew_hbm.at[e], stage_ref, sem).start()
            pltpu.make_async_copy(ew_hbm.at[e], stage_ref, sem).wait()
            wcat_ref[e] = stage_ref[...].astype(jnp.bfloat16)
        pltpu.make_async_copy(sw_hbm.at[0], stage_ref, sem).start()
        pltpu.make_async_copy(sw_hbm.at[0], stage_ref, sem).wait()
        sw0 = stage_ref[...]
        pltpu.make_async_copy(sw_hbm.at[1], stage_ref, sem).start()
        pltpu.make_async_copy(sw_hbm.at[1], stage_ref, sem).wait()
        wcat_ref[_E] = (sw0 + stage_ref[...]).astype(jnp.bfloat16)

    x = x_ref[...]

    logits = jnp.dot(x, gw_ref[...]) + gb_ref[...]
    logits_ref[...] = logits
    probs = jax.nn.softmax(logits, axis=-1)

    iota = jax.lax.broadcasted_iota(jnp.int32, probs.shape, 1)
    v1 = jnp.max(probs, axis=-1, keepdims=True)
    i1 = jnp.min(jnp.where(probs == v1, iota, _E), axis=-1, keepdims=True)
    one1 = iota == i1
    probs2 = jnp.where(one1, -jnp.inf, probs)
    v2 = jnp.max(probs2, axis=-1, keepdims=True)
    i2 = jnp.min(jnp.where(probs2 == v2, iota, _E), axis=-1, keepdims=True)
    one2 = iota == i2
    denom = v1 + v2
    combine = jnp.where(one1, v1 / denom, 0.0) + jnp.where(one2, v2 / denom, 0.0)
    combine = combine.astype(jnp.float32)

    acc = jnp.dot(combine, eb_ref[...], preferred_element_type=jnp.float32)
    acc += sb_ref[0:1, :] + sb_ref[1:2, :]

    xb = x.astype(jnp.bfloat16)
    cb = combine.astype(jnp.bfloat16)
    acc += jnp.dot(xb, wcat_ref[_E], preferred_element_type=jnp.float32)
    for e in range(_E):
        xe = xb * cb[:, e:e + 1]
        acc += jnp.dot(xe, wcat_ref[e], preferred_element_type=jnp.float32)

    out_ref[...] = acc


def kernel(x, gate_w, gate_b, expert_w, expert_b, shared_w, shared_b):
    b, s, h = x.shape
    hs = x.reshape(-1, h)
    t = hs.shape[0]

    grid = (t // _BT,)
    out, logits = pl.pallas_call(
        _moe_tile,
        grid=grid,
        in_specs=[
            pl.BlockSpec((_BT, h), lambda i: (i, 0)),
            pl.BlockSpec((h, _E), lambda i: (0, 0)),
            pl.BlockSpec((1, _E), lambda i: (0, 0)),
            pl.BlockSpec(memory_space=pl.ANY),
            pl.BlockSpec(memory_space=pl.ANY),
            pl.BlockSpec((_E, h), lambda i: (0, 0)),
            pl.BlockSpec((2, h), lambda i: (0, 0)),
        ],
        out_specs=[
            pl.BlockSpec((_BT, h), lambda i: (i, 0)),
            pl.BlockSpec((_BT, _E), lambda i: (i, 0)),
        ],
        out_shape=[
            jax.ShapeDtypeStruct((t, h), jnp.float32),
            jax.ShapeDtypeStruct((t, _E), jnp.float32),
        ],
        scratch_shapes=[
            pltpu.VMEM((_E + 1, h, h), jnp.bfloat16),
            pltpu.VMEM((h, h), jnp.float32),
            pltpu.SemaphoreType.DMA,
        ],
        compiler_params=pltpu.CompilerParams(
            dimension_semantics=("arbitrary",),
        ),
    )(hs, gate_w, gate_b.reshape(1, _E), expert_w, shared_w, expert_b,
      shared_b)
    return out.reshape(b, s, h), logits
